# baseline (speedup 1.0000x reference)
import functools

import jax
import jax.numpy as jnp
from jax import lax
from jax.experimental import pallas as pl
from jax.experimental.pallas import tpu as pltpu
from jax.experimental.pallas import tpu_sc as plsc

N = 10000
E = 320000
D = 128

NC = 2
NS = 16
NW = NC * NS
W = 96
PE = E // NW
NF = PE // W
TAIL = PE - NF * W
SLAB = 640
LAST_SLAB = N - (NS - 1) * SLAB

_mesh = plsc.VectorSubcoreMesh(core_axis_name="c", subcore_axis_name="s")


@functools.partial(
    pl.kernel,
    out_type=jax.ShapeDtypeStruct((NC, N), jnp.float32),
    mesh=_mesh,
    scratch_types=[
        pltpu.VMEM((PE,), jnp.int32),
        pltpu.VMEM((W,), jnp.int32),
        pltpu.VMEM((W,), jnp.float32),
        pltpu.VMEM((TAIL,), jnp.int32),
        pltpu.VMEM((TAIL,), jnp.float32),
        pltpu.VMEM_SHARED((N,), jnp.float32),
    ],
)
def _deg_sc(dst_hbm, zero_hbm, out_hbm, dst_all, dst_v, ones_v, dst_t, ones_t,
            cnt_sh):
    cid = lax.axis_index("c")
    sid = lax.axis_index("s")
    wid = sid * NC + cid
    e0 = pl.multiple_of(wid * PE, 8)
    pltpu.sync_copy(dst_hbm.at[pl.ds(e0, PE)], dst_all)
    for i in range(W // 16):
        ones_v[pl.ds(i * 16, 16)] = jnp.full((16,), 1.0, jnp.float32)
    ones_t[...] = jnp.full((TAIL,), 1.0, jnp.float32)

    @pl.when(sid == 0)
    def _():
        pltpu.sync_copy(zero_hbm, cnt_sh)

    plsc.subcore_barrier()

    def body(j, carry):
        for i in range(W // 16):
            dst_v[pl.ds(i * 16, 16)] = dst_all[pl.ds(j * W + i * 16, 16)]
        pltpu.sync_copy(ones_v, cnt_sh.at[dst_v], add=True)
        return carry

    lax.fori_loop(0, NF, body, 0)
    dst_t[...] = dst_all[pl.ds(NF * W, TAIL)]
    pltpu.sync_copy(ones_t, cnt_sh.at[dst_t], add=True)
    plsc.subcore_barrier()

    @pl.when(sid == 0)
    def _():
        pltpu.sync_copy(cnt_sh, out_hbm.at[cid])


@functools.partial(
    pl.kernel,
    out_type=jax.ShapeDtypeStruct((NC, N, D), jnp.float32),
    mesh=_mesh,
    scratch_types=[
        pltpu.VMEM((PE,), jnp.int32),
        pltpu.VMEM((PE,), jnp.int32),
        pltpu.VMEM((2, W), jnp.int32),
        pltpu.VMEM((2, W, D), jnp.float32),
        pltpu.VMEM((TAIL,), jnp.int32),
        pltpu.VMEM_SHARED((N, D), jnp.float32),
        pltpu.SemaphoreType.DMA,
        pltpu.SemaphoreType.DMA,
        pltpu.SemaphoreType.DMA,
    ],
)
def _agg_sc(table_hbm, src_hbm, dst_hbm, out_hbm, src_all, dst_all, dst_v,
            rows_v, dst_t, acc_sh, sem, sem_s0, sem_s1):
    cid = lax.axis_index("c")
    sid = lax.axis_index("s")
    wid = sid * NC + cid
    r0 = pl.multiple_of(sid * SLAB, 8)
    e0 = pl.multiple_of(wid * PE, 8)
    pltpu.sync_copy(src_hbm.at[pl.ds(e0, PE)], src_all)
    pltpu.sync_copy(dst_hbm.at[pl.ds(e0, PE)], dst_all)

    @pl.when(sid < NS - 1)
    def _():
        pltpu.sync_copy(table_hbm.at[pl.ds(r0, SLAB)], acc_sh.at[pl.ds(r0, SLAB)])

    @pl.when(sid == NS - 1)
    def _():
        pltpu.sync_copy(table_hbm.at[pl.ds(r0, LAST_SLAB)],
                        acc_sh.at[pl.ds(r0, LAST_SLAB)])

    plsc.subcore_barrier()

    def stage_dst(j, b):
        for i in range(W // 16):
            dst_v[b, pl.ds(i * 16, 16)] = dst_all[pl.ds(j * W + i * 16, 16)]

    def fire_gather(j, b):
        pltpu.async_copy(table_hbm.at[src_all.at[pl.ds(j * W, W)]],
                         rows_v.at[b], sem)

    sem_s = (sem_s0, sem_s1)

    def wait_gather(b):
        pltpu.make_async_copy(table_hbm.at[pl.ds(0, W)], rows_v.at[b],
                              sem).wait()

    def fire_scatter(b):
        pltpu.async_copy(rows_v.at[b], acc_sh.at[dst_v.at[b]], sem_s[b],
                         add=True)

    def wait_scatter(b):
        pltpu.make_async_copy(rows_v.at[b], acc_sh.at[dst_v.at[b]],
                              sem_s[b]).wait()

    stage_dst(0, 0)
    fire_gather(0, 0)

    def pair(j2, carry):
        for b in range(2):
            j = j2 + b
            nb = 1 - b
            wait_gather(b)

            @pl.when(j >= 1)
            def _():
                wait_scatter(nb)

            @pl.when(j + 1 < NF)
            def _():
                stage_dst(j + 1, nb)
                fire_gather(j + 1, nb)

            fire_scatter(b)
        return carry

    lax.fori_loop(0, NF // 2, lambda t, c: pair(t * 2, c), 0)
    wait_scatter(1)

    dst_t[...] = dst_all[pl.ds(NF * W, TAIL)]
    pltpu.async_copy(table_hbm.at[src_all.at[pl.ds(NF * W, TAIL)]],
                     rows_v.at[0, pl.ds(0, TAIL)], sem).wait()
    pltpu.sync_copy(rows_v.at[0, pl.ds(0, TAIL)], acc_sh.at[dst_t], add=True)

    plsc.subcore_barrier()

    @pl.when(sid < NS - 1)
    def _():
        pltpu.sync_copy(acc_sh.at[pl.ds(r0, SLAB)], out_hbm.at[cid, pl.ds(r0, SLAB)])

    @pl.when(sid == NS - 1)
    def _():
        pltpu.sync_copy(acc_sh.at[pl.ds(r0, LAST_SLAB)],
                        out_hbm.at[cid, pl.ds(r0, LAST_SLAB)])


_RB = 1024
_GRID = (N + _RB - 1) // _RB


def _first_body(c0_ref, c1_ref, x_ref, w_ref, dis_ref, hs_ref):
    deg = c0_ref[:] + c1_ref[:] + 1.0
    dis = lax.rsqrt(deg)
    dis_ref[:] = dis
    h = jnp.dot(x_ref[:], w_ref[:], preferred_element_type=jnp.float32)
    hs_ref[:] = h * dis[:, None]


def _first_tc(c0, c1, x, w):
    return pl.pallas_call(
        _first_body,
        grid=(_GRID,),
        in_specs=[
            pl.BlockSpec((_RB,), lambda i: (i,)),
            pl.BlockSpec((_RB,), lambda i: (i,)),
            pl.BlockSpec((_RB, D), lambda i: (i, 0)),
            pl.BlockSpec((D, D), lambda i: (0, 0)),
        ],
        out_specs=[
            pl.BlockSpec((_RB,), lambda i: (i,)),
            pl.BlockSpec((_RB, D), lambda i: (i, 0)),
        ],
        out_shape=[
            jax.ShapeDtypeStruct((N,), jnp.float32),
            jax.ShapeDtypeStruct((N, D), jnp.float32),
        ],
    )(c0, c1, x, w)


def _mid_body(a0_ref, a1_ref, hs_ref, dis_ref, b_ref, w_ref, out_ref):
    tot = a0_ref[:] + a1_ref[:] - hs_ref[:]
    dis = dis_ref[:]
    o1 = jnp.maximum(tot * dis[:, None] + b_ref[:][None, :], 0.0)
    h = jnp.dot(o1, w_ref[:], preferred_element_type=jnp.float32)
    out_ref[:] = h * dis[:, None]


def _mid_tc(a0, a1, hs, dis, b, w):
    return pl.pallas_call(
        _mid_body,
        grid=(_GRID,),
        in_specs=[
            pl.BlockSpec((_RB, D), lambda i: (i, 0)),
            pl.BlockSpec((_RB, D), lambda i: (i, 0)),
            pl.BlockSpec((_RB, D), lambda i: (i, 0)),
            pl.BlockSpec((_RB,), lambda i: (i,)),
            pl.BlockSpec((D,), lambda i: (0,)),
            pl.BlockSpec((D, D), lambda i: (0, 0)),
        ],
        out_specs=pl.BlockSpec((_RB, D), lambda i: (i, 0)),
        out_shape=jax.ShapeDtypeStruct((N, D), jnp.float32),
    )(a0, a1, hs, dis, b, w)


def _final_body(a0_ref, a1_ref, hs_ref, dis_ref, b_ref, out_ref):
    tot = a0_ref[:] + a1_ref[:] - hs_ref[:]
    out_ref[:] = tot * dis_ref[:][:, None] + b_ref[:][None, :]


def _final_tc(a0, a1, hs, dis, b):
    return pl.pallas_call(
        _final_body,
        grid=(_GRID,),
        in_specs=[
            pl.BlockSpec((_RB, D), lambda i: (i, 0)),
            pl.BlockSpec((_RB, D), lambda i: (i, 0)),
            pl.BlockSpec((_RB, D), lambda i: (i, 0)),
            pl.BlockSpec((_RB,), lambda i: (i,)),
            pl.BlockSpec((D,), lambda i: (0,)),
        ],
        out_specs=pl.BlockSpec((_RB, D), lambda i: (i, 0)),
        out_shape=jax.ShapeDtypeStruct((N, D), jnp.float32),
    )(a0, a1, hs, dis, b)


def kernel(x, edge_index, W1, b1, W2, b2):
    src = edge_index[0].astype(jnp.int32)
    dst = edge_index[1].astype(jnp.int32)
    zero = jnp.zeros((N,), jnp.float32)

    cnt = _deg_sc(dst, zero)
    dis, h1s = _first_tc(cnt[0], cnt[1], x, W1)
    agg1 = _agg_sc(h1s, src, dst)
    h2s = _mid_tc(agg1[0], agg1[1], h1s, dis, b1, W2)
    agg2 = _agg_sc(h2s, src, dst)
    return _final_tc(agg2[0], agg2[1], h2s, dis, b2)


# speedup vs baseline: 1.0055x; 1.0055x over previous
import functools

import jax
import jax.numpy as jnp
from jax import lax
from jax.experimental import pallas as pl
from jax.experimental.pallas import tpu as pltpu
from jax.experimental.pallas import tpu_sc as plsc

N = 10000
E = 320000
D = 128

NC = 2
NS = 16
NW = NC * NS
W = 96
PE = E // NW
NF = PE // W
TAIL = PE - NF * W
SLAB = 640
LAST_SLAB = N - (NS - 1) * SLAB

_mesh = plsc.VectorSubcoreMesh(core_axis_name="c", subcore_axis_name="s")


@functools.partial(
    pl.kernel,
    out_type=jax.ShapeDtypeStruct((NC, N), jnp.float32),
    mesh=_mesh,
    scratch_types=[
        pltpu.VMEM((PE,), jnp.int32),
        pltpu.VMEM((W,), jnp.int32),
        pltpu.VMEM((W,), jnp.float32),
        pltpu.VMEM((TAIL,), jnp.int32),
        pltpu.VMEM((TAIL,), jnp.float32),
        pltpu.VMEM_SHARED((N,), jnp.float32),
    ],
)
def _deg_sc(dst_hbm, zero_hbm, out_hbm, dst_all, dst_v, ones_v, dst_t, ones_t,
            cnt_sh):
    cid = lax.axis_index("c")
    sid = lax.axis_index("s")
    wid = sid * NC + cid
    e0 = pl.multiple_of(wid * PE, 8)
    pltpu.sync_copy(dst_hbm.at[pl.ds(e0, PE)], dst_all)
    for i in range(W // 16):
        ones_v[pl.ds(i * 16, 16)] = jnp.full((16,), 1.0, jnp.float32)
    ones_t[...] = jnp.full((TAIL,), 1.0, jnp.float32)

    @pl.when(sid == 0)
    def _():
        pltpu.sync_copy(zero_hbm, cnt_sh)

    plsc.subcore_barrier()

    def body(j, carry):
        for i in range(W // 16):
            dst_v[pl.ds(i * 16, 16)] = dst_all[pl.ds(j * W + i * 16, 16)]
        pltpu.sync_copy(ones_v, cnt_sh.at[dst_v], add=True)
        return carry

    lax.fori_loop(0, NF, body, 0)
    dst_t[...] = dst_all[pl.ds(NF * W, TAIL)]
    pltpu.sync_copy(ones_t, cnt_sh.at[dst_t], add=True)
    plsc.subcore_barrier()

    @pl.when(sid == 0)
    def _():
        pltpu.sync_copy(cnt_sh, out_hbm.at[cid])


@functools.partial(
    pl.kernel,
    out_type=jax.ShapeDtypeStruct((NC, N, D), jnp.float32),
    mesh=_mesh,
    scratch_types=[
        pltpu.VMEM((PE,), jnp.int32),
        pltpu.VMEM((PE,), jnp.int32),
        pltpu.VMEM((2, W), jnp.int32),
        pltpu.VMEM((2, W, D), jnp.float32),
        pltpu.VMEM((TAIL,), jnp.int32),
        pltpu.VMEM_SHARED((N, D), jnp.float32),
        pltpu.SemaphoreType.DMA,
        pltpu.SemaphoreType.DMA,
        pltpu.SemaphoreType.DMA,
    ],
)
def _agg_sc(table_hbm, src_hbm, dst_hbm, out_hbm, src_all, dst_all, dst_v,
            rows_v, dst_t, acc_sh, sem, sem_s0, sem_s1):
    cid = lax.axis_index("c")
    sid = lax.axis_index("s")
    wid = sid * NC + cid
    r0 = pl.multiple_of(sid * SLAB, 8)
    e0 = pl.multiple_of(wid * PE, 8)
    pltpu.sync_copy(src_hbm.at[pl.ds(e0, PE)], src_all)
    pltpu.sync_copy(dst_hbm.at[pl.ds(e0, PE)], dst_all)

    def stage_dst(j, b):
        for i in range(W // 16):
            dst_v[b, pl.ds(i * 16, 16)] = dst_all[pl.ds(j * W + i * 16, 16)]

    def fire_gather(j, b):
        pltpu.async_copy(table_hbm.at[src_all.at[pl.ds(j * W, W)]],
                         rows_v.at[b], sem)

    stage_dst(0, 0)
    fire_gather(0, 0)

    @pl.when(sid < NS - 1)
    def _():
        pltpu.sync_copy(table_hbm.at[pl.ds(r0, SLAB)], acc_sh.at[pl.ds(r0, SLAB)])

    @pl.when(sid == NS - 1)
    def _():
        pltpu.sync_copy(table_hbm.at[pl.ds(r0, LAST_SLAB)],
                        acc_sh.at[pl.ds(r0, LAST_SLAB)])

    plsc.subcore_barrier()

    sem_s = (sem_s0, sem_s1)

    def wait_gather(b):
        pltpu.make_async_copy(table_hbm.at[pl.ds(0, W)], rows_v.at[b],
                              sem).wait()

    def fire_scatter(b):
        pltpu.async_copy(rows_v.at[b], acc_sh.at[dst_v.at[b]], sem_s[b],
                         add=True)

    def wait_scatter(b):
        pltpu.make_async_copy(rows_v.at[b], acc_sh.at[dst_v.at[b]],
                              sem_s[b]).wait()

    def pair(j2, carry):
        for b in range(2):
            j = j2 + b
            nb = 1 - b
            wait_gather(b)

            @pl.when(j >= 1)
            def _():
                wait_scatter(nb)

            @pl.when(j + 1 < NF)
            def _():
                stage_dst(j + 1, nb)
                fire_gather(j + 1, nb)

            fire_scatter(b)
        return carry

    lax.fori_loop(0, NF // 2, lambda t, c: pair(t * 2, c), 0)
    wait_scatter(1)

    dst_t[...] = dst_all[pl.ds(NF * W, TAIL)]
    pltpu.async_copy(table_hbm.at[src_all.at[pl.ds(NF * W, TAIL)]],
                     rows_v.at[0, pl.ds(0, TAIL)], sem).wait()
    pltpu.sync_copy(rows_v.at[0, pl.ds(0, TAIL)], acc_sh.at[dst_t], add=True)

    plsc.subcore_barrier()

    @pl.when(sid < NS - 1)
    def _():
        pltpu.sync_copy(acc_sh.at[pl.ds(r0, SLAB)], out_hbm.at[cid, pl.ds(r0, SLAB)])

    @pl.when(sid == NS - 1)
    def _():
        pltpu.sync_copy(acc_sh.at[pl.ds(r0, LAST_SLAB)],
                        out_hbm.at[cid, pl.ds(r0, LAST_SLAB)])


_RB = 1024
_GRID = (N + _RB - 1) // _RB


def _mm_body(x_ref, w_ref, h_ref):
    h_ref[:] = jnp.dot(x_ref[:], w_ref[:], preferred_element_type=jnp.float32)


def _mm_tc(x, w):
    return pl.pallas_call(
        _mm_body,
        grid=(_GRID,),
        in_specs=[
            pl.BlockSpec((_RB, D), lambda i: (i, 0)),
            pl.BlockSpec((D, D), lambda i: (0, 0)),
        ],
        out_specs=pl.BlockSpec((_RB, D), lambda i: (i, 0)),
        out_shape=jax.ShapeDtypeStruct((N, D), jnp.float32),
    )(x, w)


def _scale_body(c0_ref, c1_ref, h_ref, dis_ref, hs_ref):
    deg = c0_ref[:] + c1_ref[:] + 1.0
    dis = lax.rsqrt(deg)
    dis_ref[:] = dis
    hs_ref[:] = h_ref[:] * dis[:, None]


def _scale_tc(c0, c1, h):
    return pl.pallas_call(
        _scale_body,
        grid=(_GRID,),
        in_specs=[
            pl.BlockSpec((_RB,), lambda i: (i,)),
            pl.BlockSpec((_RB,), lambda i: (i,)),
            pl.BlockSpec((_RB, D), lambda i: (i, 0)),
        ],
        out_specs=[
            pl.BlockSpec((_RB,), lambda i: (i,)),
            pl.BlockSpec((_RB, D), lambda i: (i, 0)),
        ],
        out_shape=[
            jax.ShapeDtypeStruct((N,), jnp.float32),
            jax.ShapeDtypeStruct((N, D), jnp.float32),
        ],
    )(c0, c1, h)


def _mid_body(a0_ref, a1_ref, hs_ref, dis_ref, b_ref, w_ref, out_ref):
    tot = a0_ref[:] + a1_ref[:] - hs_ref[:]
    dis = dis_ref[:]
    o1 = jnp.maximum(tot * dis[:, None] + b_ref[:][None, :], 0.0)
    h = jnp.dot(o1, w_ref[:], preferred_element_type=jnp.float32)
    out_ref[:] = h * dis[:, None]


def _mid_tc(a0, a1, hs, dis, b, w):
    return pl.pallas_call(
        _mid_body,
        grid=(_GRID,),
        in_specs=[
            pl.BlockSpec((_RB, D), lambda i: (i, 0)),
            pl.BlockSpec((_RB, D), lambda i: (i, 0)),
            pl.BlockSpec((_RB, D), lambda i: (i, 0)),
            pl.BlockSpec((_RB,), lambda i: (i,)),
            pl.BlockSpec((D,), lambda i: (0,)),
            pl.BlockSpec((D, D), lambda i: (0, 0)),
        ],
        out_specs=pl.BlockSpec((_RB, D), lambda i: (i, 0)),
        out_shape=jax.ShapeDtypeStruct((N, D), jnp.float32),
    )(a0, a1, hs, dis, b, w)


def _final_body(a0_ref, a1_ref, hs_ref, dis_ref, b_ref, out_ref):
    tot = a0_ref[:] + a1_ref[:] - hs_ref[:]
    out_ref[:] = tot * dis_ref[:][:, None] + b_ref[:][None, :]


def _final_tc(a0, a1, hs, dis, b):
    return pl.pallas_call(
        _final_body,
        grid=(_GRID,),
        in_specs=[
            pl.BlockSpec((_RB, D), lambda i: (i, 0)),
            pl.BlockSpec((_RB, D), lambda i: (i, 0)),
            pl.BlockSpec((_RB, D), lambda i: (i, 0)),
            pl.BlockSpec((_RB,), lambda i: (i,)),
            pl.BlockSpec((D,), lambda i: (0,)),
        ],
        out_specs=pl.BlockSpec((_RB, D), lambda i: (i, 0)),
        out_shape=jax.ShapeDtypeStruct((N, D), jnp.float32),
    )(a0, a1, hs, dis, b)


def kernel(x, edge_index, W1, b1, W2, b2):
    src = edge_index[0].astype(jnp.int32)
    dst = edge_index[1].astype(jnp.int32)
    zero = jnp.zeros((N,), jnp.float32)

    h1 = _mm_tc(x, W1)
    cnt = _deg_sc(dst, zero)
    dis, h1s = _scale_tc(cnt[0], cnt[1], h1)
    agg1 = _agg_sc(h1s, src, dst)
    h2s = _mid_tc(agg1[0], agg1[1], h1s, dis, b1, W2)
    agg2 = _agg_sc(h2s, src, dst)
    return _final_tc(agg2[0], agg2[1], h2s, dis, b2)


# speedup vs baseline: 1.0808x; 1.0748x over previous
import functools

import jax
import jax.numpy as jnp
from jax import lax
from jax.experimental import pallas as pl
from jax.experimental.pallas import tpu as pltpu
from jax.experimental.pallas import tpu_sc as plsc

N = 10000
E = 320000
D = 128

NC = 2
NS = 16
NW = NC * NS
W = 96
PE = E // NW
NF = PE // W
TAIL = PE - NF * W
SLAB = 640
LAST_SLAB = N - (NS - 1) * SLAB

_mesh = plsc.VectorSubcoreMesh(core_axis_name="c", subcore_axis_name="s")


@functools.partial(
    pl.kernel,
    out_type=jax.ShapeDtypeStruct((NC, N), jnp.float32),
    mesh=_mesh,
    scratch_types=[
        pltpu.VMEM((PE,), jnp.int32),
        pltpu.VMEM((W,), jnp.int32),
        pltpu.VMEM((W,), jnp.float32),
        pltpu.VMEM((TAIL,), jnp.int32),
        pltpu.VMEM((TAIL,), jnp.float32),
        pltpu.VMEM((SLAB,), jnp.float32),
        pltpu.VMEM_SHARED((N,), jnp.float32),
    ],
)
def _deg_sc(ei_hbm, out_hbm, dst_all, dst_v, ones_v, dst_t, ones_t, zeros_v,
            cnt_sh):
    cid = lax.axis_index("c")
    sid = lax.axis_index("s")
    wid = sid * NC + cid
    e0 = pl.multiple_of(wid * PE, 8)
    pltpu.sync_copy(ei_hbm.at[pl.ds(E + e0, PE)], dst_all)
    for i in range(W // 16):
        ones_v[pl.ds(i * 16, 16)] = jnp.full((16,), 1.0, jnp.float32)
    ones_t[...] = jnp.full((TAIL,), 1.0, jnp.float32)

    for i in range(SLAB // 16):
        zeros_v[pl.ds(i * 16, 16)] = jnp.zeros((16,), jnp.float32)
    r0 = pl.multiple_of(sid * SLAB, 8)

    @pl.when(sid < NS - 1)
    def _():
        pltpu.sync_copy(zeros_v, cnt_sh.at[pl.ds(r0, SLAB)])

    @pl.when(sid == NS - 1)
    def _():
        pltpu.sync_copy(zeros_v.at[pl.ds(0, LAST_SLAB)],
                        cnt_sh.at[pl.ds(r0, LAST_SLAB)])

    plsc.subcore_barrier()

    def body(j, carry):
        for i in range(W // 16):
            dst_v[pl.ds(i * 16, 16)] = dst_all[pl.ds(j * W + i * 16, 16)]
        pltpu.sync_copy(ones_v, cnt_sh.at[dst_v], add=True)
        return carry

    lax.fori_loop(0, NF, body, 0)
    dst_t[...] = dst_all[pl.ds(NF * W, TAIL)]
    pltpu.sync_copy(ones_t, cnt_sh.at[dst_t], add=True)
    plsc.subcore_barrier()

    @pl.when(sid == 0)
    def _():
        pltpu.sync_copy(cnt_sh, out_hbm.at[cid])


@functools.partial(
    pl.kernel,
    out_type=jax.ShapeDtypeStruct((NC, N, D), jnp.float32),
    mesh=_mesh,
    scratch_types=[
        pltpu.VMEM((PE,), jnp.int32),
        pltpu.VMEM((PE,), jnp.int32),
        pltpu.VMEM((2, W), jnp.int32),
        pltpu.VMEM((2, W, D), jnp.float32),
        pltpu.VMEM((TAIL,), jnp.int32),
        pltpu.VMEM_SHARED((N, D), jnp.float32),
        pltpu.SemaphoreType.DMA,
        pltpu.SemaphoreType.DMA,
        pltpu.SemaphoreType.DMA,
    ],
)
def _agg_sc(table_hbm, ei_hbm, out_hbm, src_all, dst_all, dst_v,
            rows_v, dst_t, acc_sh, sem, sem_s0, sem_s1):
    cid = lax.axis_index("c")
    sid = lax.axis_index("s")
    wid = sid * NC + cid
    r0 = pl.multiple_of(sid * SLAB, 8)
    e0 = pl.multiple_of(wid * PE, 8)
    pltpu.sync_copy(ei_hbm.at[pl.ds(e0, PE)], src_all)
    pltpu.sync_copy(ei_hbm.at[pl.ds(E + e0, PE)], dst_all)

    def stage_dst(j, b):
        for i in range(W // 16):
            dst_v[b, pl.ds(i * 16, 16)] = dst_all[pl.ds(j * W + i * 16, 16)]

    def fire_gather(j, b):
        pltpu.async_copy(table_hbm.at[src_all.at[pl.ds(j * W, W)]],
                         rows_v.at[b], sem)

    stage_dst(0, 0)
    fire_gather(0, 0)

    @pl.when(sid < NS - 1)
    def _():
        pltpu.sync_copy(table_hbm.at[pl.ds(r0, SLAB)], acc_sh.at[pl.ds(r0, SLAB)])

    @pl.when(sid == NS - 1)
    def _():
        pltpu.sync_copy(table_hbm.at[pl.ds(r0, LAST_SLAB)],
                        acc_sh.at[pl.ds(r0, LAST_SLAB)])

    plsc.subcore_barrier()

    sem_s = (sem_s0, sem_s1)

    def wait_gather(b):
        pltpu.make_async_copy(table_hbm.at[pl.ds(0, W)], rows_v.at[b],
                              sem).wait()

    def fire_scatter(b):
        pltpu.async_copy(rows_v.at[b], acc_sh.at[dst_v.at[b]], sem_s[b],
                         add=True)

    def wait_scatter(b):
        pltpu.make_async_copy(rows_v.at[b], acc_sh.at[dst_v.at[b]],
                              sem_s[b]).wait()

    def pair(j2, carry):
        for b in range(2):
            j = j2 + b
            nb = 1 - b
            wait_gather(b)

            @pl.when(j >= 1)
            def _():
                wait_scatter(nb)

            @pl.when(j + 1 < NF)
            def _():
                stage_dst(j + 1, nb)
                fire_gather(j + 1, nb)

            fire_scatter(b)
        return carry

    lax.fori_loop(0, NF // 2, lambda t, c: pair(t * 2, c), 0)
    wait_scatter(1)

    dst_t[...] = dst_all[pl.ds(NF * W, TAIL)]
    pltpu.async_copy(table_hbm.at[src_all.at[pl.ds(NF * W, TAIL)]],
                     rows_v.at[0, pl.ds(0, TAIL)], sem).wait()
    pltpu.sync_copy(rows_v.at[0, pl.ds(0, TAIL)], acc_sh.at[dst_t], add=True)

    plsc.subcore_barrier()

    @pl.when(sid < NS - 1)
    def _():
        pltpu.sync_copy(acc_sh.at[pl.ds(r0, SLAB)], out_hbm.at[cid, pl.ds(r0, SLAB)])

    @pl.when(sid == NS - 1)
    def _():
        pltpu.sync_copy(acc_sh.at[pl.ds(r0, LAST_SLAB)],
                        out_hbm.at[cid, pl.ds(r0, LAST_SLAB)])


_RB = 1024
_GRID = (N + _RB - 1) // _RB


def _mm_body(x_ref, w_ref, h_ref):
    h_ref[:] = jnp.dot(x_ref[:], w_ref[:], preferred_element_type=jnp.float32)


def _mm_tc(x, w):
    return pl.pallas_call(
        _mm_body,
        grid=(_GRID,),
        in_specs=[
            pl.BlockSpec((_RB, D), lambda i: (i, 0)),
            pl.BlockSpec((D, D), lambda i: (0, 0)),
        ],
        out_specs=pl.BlockSpec((_RB, D), lambda i: (i, 0)),
        out_shape=jax.ShapeDtypeStruct((N, D), jnp.float32),
    )(x, w)


def _scale_body(c0_ref, c1_ref, h_ref, dis_ref, hs_ref):
    deg = c0_ref[:] + c1_ref[:] + 1.0
    dis = lax.rsqrt(deg)
    dis_ref[:] = dis
    hs_ref[:] = h_ref[:] * dis[:, None]


def _scale_tc(c0, c1, h):
    return pl.pallas_call(
        _scale_body,
        grid=(_GRID,),
        in_specs=[
            pl.BlockSpec((_RB,), lambda i: (i,)),
            pl.BlockSpec((_RB,), lambda i: (i,)),
            pl.BlockSpec((_RB, D), lambda i: (i, 0)),
        ],
        out_specs=[
            pl.BlockSpec((_RB,), lambda i: (i,)),
            pl.BlockSpec((_RB, D), lambda i: (i, 0)),
        ],
        out_shape=[
            jax.ShapeDtypeStruct((N,), jnp.float32),
            jax.ShapeDtypeStruct((N, D), jnp.float32),
        ],
    )(c0, c1, h)


def _mid_body(a0_ref, a1_ref, hs_ref, dis_ref, b_ref, w_ref, out_ref):
    tot = a0_ref[0] + a1_ref[0] - hs_ref[:]
    dis = dis_ref[:]
    o1 = jnp.maximum(tot * dis[:, None] + b_ref[:][None, :], 0.0)
    h = jnp.dot(o1, w_ref[:], preferred_element_type=jnp.float32)
    out_ref[:] = h * dis[:, None]


def _mid_tc(agg, hs, dis, b, w):
    return pl.pallas_call(
        _mid_body,
        grid=(_GRID,),
        in_specs=[
            pl.BlockSpec((1, _RB, D), lambda i: (0, i, 0)),
            pl.BlockSpec((1, _RB, D), lambda i: (1, i, 0)),
            pl.BlockSpec((_RB, D), lambda i: (i, 0)),
            pl.BlockSpec((_RB,), lambda i: (i,)),
            pl.BlockSpec((D,), lambda i: (0,)),
            pl.BlockSpec((D, D), lambda i: (0, 0)),
        ],
        out_specs=pl.BlockSpec((_RB, D), lambda i: (i, 0)),
        out_shape=jax.ShapeDtypeStruct((N, D), jnp.float32),
    )(agg, agg, hs, dis, b, w)


def _final_body(a0_ref, a1_ref, hs_ref, dis_ref, b_ref, out_ref):
    tot = a0_ref[0] + a1_ref[0] - hs_ref[:]
    out_ref[:] = tot * dis_ref[:][:, None] + b_ref[:][None, :]


def _final_tc(agg, hs, dis, b):
    return pl.pallas_call(
        _final_body,
        grid=(_GRID,),
        in_specs=[
            pl.BlockSpec((1, _RB, D), lambda i: (0, i, 0)),
            pl.BlockSpec((1, _RB, D), lambda i: (1, i, 0)),
            pl.BlockSpec((_RB, D), lambda i: (i, 0)),
            pl.BlockSpec((_RB,), lambda i: (i,)),
            pl.BlockSpec((D,), lambda i: (0,)),
        ],
        out_specs=pl.BlockSpec((_RB, D), lambda i: (i, 0)),
        out_shape=jax.ShapeDtypeStruct((N, D), jnp.float32),
    )(agg, agg, hs, dis, b)


def kernel(x, edge_index, W1, b1, W2, b2):
    ei = edge_index.astype(jnp.int32).reshape(-1)

    h1 = _mm_tc(x, W1)
    cnt = _deg_sc(ei)
    dis, h1s = _scale_tc(cnt[0], cnt[1], h1)
    agg1 = _agg_sc(h1s, ei)
    h2s = _mid_tc(agg1, h1s, dis, b1, W2)
    agg2 = _agg_sc(h2s, ei)
    return _final_tc(agg2, h2s, dis, b2)


# speedup vs baseline: 1.4783x; 1.3679x over previous
import functools

import jax
import jax.numpy as jnp
from jax import lax
from jax.experimental import pallas as pl
from jax.experimental.pallas import tpu as pltpu
from jax.experimental.pallas import tpu_sc as plsc

N = 10000
E = 320000
D = 128

NC = 2
NS = 16
NW = NC * NS
W = 96
PE = E // NW
NF = PE // W
TAIL = PE - NF * W
SLAB = 640
LAST_SLAB = N - (NS - 1) * SLAB

_mesh = plsc.VectorSubcoreMesh(core_axis_name="c", subcore_axis_name="s")


@functools.partial(
    pl.kernel,
    out_type=jax.ShapeDtypeStruct((NC, N), jnp.float32),
    mesh=_mesh,
    scratch_types=[
        pltpu.VMEM((PE,), jnp.int32),
        pltpu.VMEM((W,), jnp.int32),
        pltpu.VMEM((W,), jnp.float32),
        pltpu.VMEM((TAIL,), jnp.int32),
        pltpu.VMEM((TAIL,), jnp.float32),
        pltpu.VMEM((SLAB,), jnp.float32),
        pltpu.VMEM_SHARED((N,), jnp.float32),
    ],
)
def _deg_sc(ei_hbm, out_hbm, dst_all, dst_v, ones_v, dst_t, ones_t, zeros_v,
            cnt_sh):
    cid = lax.axis_index("c")
    sid = lax.axis_index("s")
    wid = sid * NC + cid
    e0 = pl.multiple_of(wid * PE, 8)
    pltpu.sync_copy(ei_hbm.at[pl.ds(E + e0, PE)], dst_all)
    for i in range(W // 16):
        ones_v[pl.ds(i * 16, 16)] = jnp.full((16,), 1.0, jnp.float32)
    ones_t[...] = jnp.full((TAIL,), 1.0, jnp.float32)

    for i in range(SLAB // 16):
        zeros_v[pl.ds(i * 16, 16)] = jnp.zeros((16,), jnp.float32)
    r0 = pl.multiple_of(sid * SLAB, 8)

    @pl.when(sid < NS - 1)
    def _():
        pltpu.sync_copy(zeros_v, cnt_sh.at[pl.ds(r0, SLAB)])

    @pl.when(sid == NS - 1)
    def _():
        pltpu.sync_copy(zeros_v.at[pl.ds(0, LAST_SLAB)],
                        cnt_sh.at[pl.ds(r0, LAST_SLAB)])

    plsc.subcore_barrier()

    def body(j, carry):
        for i in range(W // 16):
            dst_v[pl.ds(i * 16, 16)] = dst_all[pl.ds(j * W + i * 16, 16)]
        pltpu.sync_copy(ones_v, cnt_sh.at[dst_v], add=True)
        return carry

    lax.fori_loop(0, NF, body, 0)
    dst_t[...] = dst_all[pl.ds(NF * W, TAIL)]
    pltpu.sync_copy(ones_t, cnt_sh.at[dst_t], add=True)
    plsc.subcore_barrier()

    @pl.when(sid == 0)
    def _():
        pltpu.sync_copy(cnt_sh, out_hbm.at[cid])


NB = 4


@functools.partial(
    pl.kernel,
    out_type=jax.ShapeDtypeStruct((NC, N, D), jnp.float32),
    mesh=_mesh,
    scratch_types=[
        pltpu.VMEM((NB, W), jnp.int32),
        pltpu.VMEM((NB, W), jnp.int32),
        pltpu.VMEM((NB, W, D), jnp.float32),
        pltpu.VMEM((TAIL,), jnp.int32),
        pltpu.VMEM_SHARED((N, D), jnp.float32),
        [pltpu.SemaphoreType.DMA] * NB,
        [pltpu.SemaphoreType.DMA] * NB,
        [pltpu.SemaphoreType.DMA] * NB,
    ],
)
def _agg_sc(table_hbm, ei_hbm, out_hbm, src_w, dst_w, rows_v, dst_t, acc_sh,
            sem_i, sem_g, sem_s):
    cid = lax.axis_index("c")
    sid = lax.axis_index("s")
    wid = sid * NC + cid
    r0 = pl.multiple_of(sid * SLAB, 8)
    e0 = pl.multiple_of(wid * PE, 8)

    def fire_idx(j, b):
        base = pl.multiple_of(e0 + j * W, 8)
        pltpu.async_copy(ei_hbm.at[pl.ds(base, W)], src_w.at[b], sem_i[b])
        pltpu.async_copy(ei_hbm.at[pl.ds(E + base, W)], dst_w.at[b], sem_i[b])

    def wait_idx(b):
        pltpu.make_async_copy(ei_hbm.at[pl.ds(0, W)], src_w.at[b],
                              sem_i[b]).wait()
        pltpu.make_async_copy(ei_hbm.at[pl.ds(0, W)], dst_w.at[b],
                              sem_i[b]).wait()

    def fire_gather(b):
        pltpu.async_copy(table_hbm.at[src_w.at[b]], rows_v.at[b], sem_g[b])

    def wait_gather(b):
        pltpu.make_async_copy(table_hbm.at[pl.ds(0, W)], rows_v.at[b],
                              sem_g[b]).wait()

    def fire_scatter(b):
        pltpu.async_copy(rows_v.at[b], acc_sh.at[dst_w.at[b]], sem_s[b],
                         add=True)

    def wait_scatter(b):
        pltpu.make_async_copy(rows_v.at[b], acc_sh.at[dst_w.at[b]],
                              sem_s[b]).wait()

    fire_idx(0, 0)
    fire_idx(1, 1)
    fire_idx(2, 2)
    wait_idx(0)
    fire_gather(0)
    wait_idx(1)
    fire_gather(1)

    @pl.when(sid < NS - 1)
    def _():
        pltpu.sync_copy(table_hbm.at[pl.ds(r0, SLAB)], acc_sh.at[pl.ds(r0, SLAB)])

    @pl.when(sid == NS - 1)
    def _():
        pltpu.sync_copy(table_hbm.at[pl.ds(r0, LAST_SLAB)],
                        acc_sh.at[pl.ds(r0, LAST_SLAB)])

    plsc.subcore_barrier()

    def step(j, b):
        wait_gather(b)
        fire_scatter(b)

        @pl.when(j >= 1)
        def _():
            wait_scatter((b - 1) % NB)

        @pl.when(j + 3 < NF)
        def _():
            fire_idx(j + 3, (b + 3) % NB)

        @pl.when(j + 2 < NF)
        def _():
            wait_idx((b + 2) % NB)
            fire_gather((b + 2) % NB)

    def quad(g, carry):
        for b in range(NB):
            step(g * NB + b, b)
        return carry

    lax.fori_loop(0, NF // NB, quad, 0)
    wait_scatter((NF - 1) % NB)

    base_t = pl.multiple_of(e0 + NF * W, 8)
    pltpu.sync_copy(ei_hbm.at[pl.ds(base_t, TAIL)], src_w.at[0, pl.ds(0, TAIL)])
    pltpu.sync_copy(ei_hbm.at[pl.ds(E + base_t, TAIL)], dst_t)
    pltpu.async_copy(table_hbm.at[src_w.at[0, pl.ds(0, TAIL)]],
                     rows_v.at[0, pl.ds(0, TAIL)], sem_g[0]).wait()
    pltpu.sync_copy(rows_v.at[0, pl.ds(0, TAIL)], acc_sh.at[dst_t], add=True)

    plsc.subcore_barrier()

    @pl.when(sid < NS - 1)
    def _():
        pltpu.sync_copy(acc_sh.at[pl.ds(r0, SLAB)], out_hbm.at[cid, pl.ds(r0, SLAB)])

    @pl.when(sid == NS - 1)
    def _():
        pltpu.sync_copy(acc_sh.at[pl.ds(r0, LAST_SLAB)],
                        out_hbm.at[cid, pl.ds(r0, LAST_SLAB)])


_RB = 1024
_GRID = (N + _RB - 1) // _RB


def _mm_body(x_ref, w_ref, h_ref):
    h_ref[:] = jnp.dot(x_ref[:], w_ref[:], preferred_element_type=jnp.float32)


def _mm_tc(x, w):
    return pl.pallas_call(
        _mm_body,
        grid=(_GRID,),
        in_specs=[
            pl.BlockSpec((_RB, D), lambda i: (i, 0)),
            pl.BlockSpec((D, D), lambda i: (0, 0)),
        ],
        out_specs=pl.BlockSpec((_RB, D), lambda i: (i, 0)),
        out_shape=jax.ShapeDtypeStruct((N, D), jnp.float32),
    )(x, w)


def _scale_body(c0_ref, c1_ref, h_ref, dis_ref, hs_ref):
    deg = c0_ref[:] + c1_ref[:] + 1.0
    dis = lax.rsqrt(deg)
    dis_ref[:] = dis
    hs_ref[:] = h_ref[:] * dis[:, None]


def _scale_tc(c0, c1, h):
    return pl.pallas_call(
        _scale_body,
        grid=(_GRID,),
        in_specs=[
            pl.BlockSpec((_RB,), lambda i: (i,)),
            pl.BlockSpec((_RB,), lambda i: (i,)),
            pl.BlockSpec((_RB, D), lambda i: (i, 0)),
        ],
        out_specs=[
            pl.BlockSpec((_RB,), lambda i: (i,)),
            pl.BlockSpec((_RB, D), lambda i: (i, 0)),
        ],
        out_shape=[
            jax.ShapeDtypeStruct((N,), jnp.float32),
            jax.ShapeDtypeStruct((N, D), jnp.float32),
        ],
    )(c0, c1, h)


def _mid_body(a0_ref, a1_ref, hs_ref, dis_ref, b_ref, w_ref, out_ref):
    tot = a0_ref[0] + a1_ref[0] - hs_ref[:]
    dis = dis_ref[:]
    o1 = jnp.maximum(tot * dis[:, None] + b_ref[:][None, :], 0.0)
    h = jnp.dot(o1, w_ref[:], preferred_element_type=jnp.float32)
    out_ref[:] = h * dis[:, None]


def _mid_tc(agg, hs, dis, b, w):
    return pl.pallas_call(
        _mid_body,
        grid=(_GRID,),
        in_specs=[
            pl.BlockSpec((1, _RB, D), lambda i: (0, i, 0)),
            pl.BlockSpec((1, _RB, D), lambda i: (1, i, 0)),
            pl.BlockSpec((_RB, D), lambda i: (i, 0)),
            pl.BlockSpec((_RB,), lambda i: (i,)),
            pl.BlockSpec((D,), lambda i: (0,)),
            pl.BlockSpec((D, D), lambda i: (0, 0)),
        ],
        out_specs=pl.BlockSpec((_RB, D), lambda i: (i, 0)),
        out_shape=jax.ShapeDtypeStruct((N, D), jnp.float32),
    )(agg, agg, hs, dis, b, w)


def _final_body(a0_ref, a1_ref, hs_ref, dis_ref, b_ref, out_ref):
    tot = a0_ref[0] + a1_ref[0] - hs_ref[:]
    out_ref[:] = tot * dis_ref[:][:, None] + b_ref[:][None, :]


def _final_tc(agg, hs, dis, b):
    return pl.pallas_call(
        _final_body,
        grid=(_GRID,),
        in_specs=[
            pl.BlockSpec((1, _RB, D), lambda i: (0, i, 0)),
            pl.BlockSpec((1, _RB, D), lambda i: (1, i, 0)),
            pl.BlockSpec((_RB, D), lambda i: (i, 0)),
            pl.BlockSpec((_RB,), lambda i: (i,)),
            pl.BlockSpec((D,), lambda i: (0,)),
        ],
        out_specs=pl.BlockSpec((_RB, D), lambda i: (i, 0)),
        out_shape=jax.ShapeDtypeStruct((N, D), jnp.float32),
    )(agg, agg, hs, dis, b)


def kernel(x, edge_index, W1, b1, W2, b2):
    ei = edge_index.astype(jnp.int32).reshape(-1)

    h1 = _mm_tc(x, W1)
    cnt = _deg_sc(ei)
    dis, h1s = _scale_tc(cnt[0], cnt[1], h1)
    agg1 = _agg_sc(h1s, ei)
    h2s = _mid_tc(agg1, h1s, dis, b1, W2)
    agg2 = _agg_sc(h2s, ei)
    return _final_tc(agg2, h2s, dis, b2)
